# baseline (speedup 1.0000x reference)
import jax
import jax.numpy as jnp
from jax import lax
from jax.experimental import pallas as pl
from jax.experimental.pallas import tpu as pltpu
from jax.experimental.pallas import tpu_sc as plsc

_HBLK = 8

_NW = 32
_NBUF = 4
_CHUNK_ROWS = 248
_LOOKAHEAD = 3


def _copy_body(x_ref, o_ref):
    o_ref[...] = x_ref[...]


def _tc_copy(x):
    B, H, L, D = x.shape
    spec = pl.BlockSpec((1, _HBLK, L, D), lambda b, h: (b, h, 0, 0))
    return pl.pallas_call(
        _copy_body,
        grid=(B, H // _HBLK),
        in_specs=[spec],
        out_specs=spec,
        out_shape=jax.ShapeDtypeStruct(x.shape, x.dtype),
        compiler_params=pltpu.CompilerParams(
            dimension_semantics=("arbitrary", "arbitrary"),
        ),
    )(x)


def _sc_copy(x):
    shape = x.shape
    rows = x.size // 128
    x2 = x.reshape(rows, 128)
    rows_per_w = rows // _NW
    sizes = [_CHUNK_ROWS] * (rows_per_w // _CHUNK_ROWS)
    if rows_per_w % _CHUNK_ROWS:
        sizes.append(rows_per_w % _CHUNK_ROWS)
    offs = [0]
    for s in sizes[:-1]:
        offs.append(offs[-1] + s)
    n = len(sizes)

    mesh = plsc.VectorSubcoreMesh(core_axis_name="c", subcore_axis_name="s")

    def run(x2):
        @pl.kernel(
            out_type=jax.ShapeDtypeStruct((rows, 128), jnp.float32),
            mesh=mesh,
            scratch_types=(
                [pltpu.VMEM((_CHUNK_ROWS, 128), jnp.float32)] * _NBUF
                + [pltpu.SemaphoreType.DMA] * (2 * _NBUF)
            ),
        )
        def sc_copy_kernel(in_hbm, out_hbm, *scratch):
            bufs = scratch[:_NBUF]
            gsems = scratch[_NBUF:2 * _NBUF]
            ssems = scratch[2 * _NBUF:]
            wid = lax.axis_index("s") * 2 + lax.axis_index("c")
            base = wid * rows_per_w

            def src(c):
                return in_hbm.at[pl.ds(base + offs[c], sizes[c])]

            def dst(c):
                return out_hbm.at[pl.ds(base + offs[c], sizes[c])]

            def buf(c):
                b = bufs[c % _NBUF]
                if sizes[c] == _CHUNK_ROWS:
                    return b
                return b.at[pl.ds(0, sizes[c])]

            ins = [None] * n
            outs = [None] * n
            out_waited = [False] * n

            for c in range(min(_LOOKAHEAD, n)):
                ins[c] = pltpu.async_copy(src(c), buf(c), gsems[c % _NBUF])
            for c in range(n):
                ins[c].wait()
                outs[c] = pltpu.async_copy(buf(c), dst(c), ssems[c % _NBUF])
                nxt = c + _LOOKAHEAD
                if nxt < n:
                    if nxt >= _NBUF:
                        outs[nxt - _NBUF].wait()
                        out_waited[nxt - _NBUF] = True
                    ins[nxt] = pltpu.async_copy(
                        src(nxt), buf(nxt), gsems[nxt % _NBUF]
                    )
            for c in range(n):
                if not out_waited[c]:
                    outs[c].wait()

        return sc_copy_kernel(x2)

    return run(x2).reshape(shape)


def kernel(new_k, new_v):
    out_v = _sc_copy(new_v)
    out_k = _tc_copy(new_k)
    return (out_k, out_v)


# speedup vs baseline: 1.1543x; 1.1543x over previous
import jax
import jax.numpy as jnp
from jax.experimental import pallas as pl
from jax.experimental.pallas import tpu as pltpu

_CHUNK_ROWS = 16384
_NBUF = 4
_LOOKAHEAD = 3


def _copy_body(k_ref, v_ref, ok_ref, ov_ref, *scratch):
    bufs = scratch[:_NBUF]
    gsem = scratch[_NBUF]
    ssem = scratch[_NBUF + 1]
    rows = k_ref.shape[0]
    per_tensor = rows // _CHUNK_ROWS
    n = 2 * per_tensor

    def src(c):
        ref = k_ref if c < per_tensor else v_ref
        return ref.at[pl.ds((c % per_tensor) * _CHUNK_ROWS, _CHUNK_ROWS)]

    def dst(c):
        ref = ok_ref if c < per_tensor else ov_ref
        return ref.at[pl.ds((c % per_tensor) * _CHUNK_ROWS, _CHUNK_ROWS)]

    ins = [None] * n
    outs = [None] * n
    out_waited = [False] * n

    for c in range(min(_LOOKAHEAD, n)):
        ins[c] = pltpu.make_async_copy(src(c), bufs[c % _NBUF], gsem.at[c % _NBUF])
        ins[c].start()
    for c in range(n):
        ins[c].wait()
        outs[c] = pltpu.make_async_copy(bufs[c % _NBUF], dst(c), ssem.at[c % _NBUF])
        outs[c].start()
        nxt = c + _LOOKAHEAD
        if nxt < n:
            if nxt >= _NBUF:
                outs[nxt - _NBUF].wait()
                out_waited[nxt - _NBUF] = True
            ins[nxt] = pltpu.make_async_copy(
                src(nxt), bufs[nxt % _NBUF], gsem.at[nxt % _NBUF]
            )
            ins[nxt].start()
    for c in range(n):
        if not out_waited[c]:
            outs[c].wait()


def kernel(new_k, new_v):
    shape = new_k.shape
    rows = new_k.size // 128
    k2 = new_k.reshape(rows, 128)
    v2 = new_v.reshape(rows, 128)
    out2 = pl.pallas_call(
        _copy_body,
        in_specs=[
            pl.BlockSpec(memory_space=pl.ANY),
            pl.BlockSpec(memory_space=pl.ANY),
        ],
        out_specs=[
            pl.BlockSpec(memory_space=pl.ANY),
            pl.BlockSpec(memory_space=pl.ANY),
        ],
        out_shape=(
            jax.ShapeDtypeStruct((rows, 128), jnp.float32),
            jax.ShapeDtypeStruct((rows, 128), jnp.float32),
        ),
        scratch_shapes=(
            [pltpu.VMEM((_CHUNK_ROWS, 128), jnp.float32)] * _NBUF
            + [pltpu.SemaphoreType.DMA((_NBUF,)), pltpu.SemaphoreType.DMA((_NBUF,))]
        ),
    )(k2, v2)
    return (out2[0].reshape(shape), out2[1].reshape(shape))


# speedup vs baseline: 1.1545x; 1.0001x over previous
import jax
import jax.numpy as jnp
from jax.experimental import pallas as pl
from jax.experimental.pallas import tpu as pltpu

_CHUNK_ROWS = 16384
_NBUF = 6
_LOOKAHEAD = 5


def _copy_body(k_ref, v_ref, ok_ref, ov_ref, *scratch):
    bufs = scratch[:_NBUF]
    gsem = scratch[_NBUF]
    ssem = scratch[_NBUF + 1]
    rows = k_ref.shape[0]
    per_tensor = rows // _CHUNK_ROWS
    n = 2 * per_tensor

    def src(c):
        ref = k_ref if c < per_tensor else v_ref
        return ref.at[pl.ds((c % per_tensor) * _CHUNK_ROWS, _CHUNK_ROWS)]

    def dst(c):
        ref = ok_ref if c < per_tensor else ov_ref
        return ref.at[pl.ds((c % per_tensor) * _CHUNK_ROWS, _CHUNK_ROWS)]

    ins = [None] * n
    outs = [None] * n
    out_waited = [False] * n

    for c in range(min(_LOOKAHEAD, n)):
        ins[c] = pltpu.make_async_copy(src(c), bufs[c % _NBUF], gsem.at[c % _NBUF])
        ins[c].start()
    for c in range(n):
        ins[c].wait()
        outs[c] = pltpu.make_async_copy(bufs[c % _NBUF], dst(c), ssem.at[c % _NBUF])
        outs[c].start()
        nxt = c + _LOOKAHEAD
        if nxt < n:
            if nxt >= _NBUF:
                outs[nxt - _NBUF].wait()
                out_waited[nxt - _NBUF] = True
            ins[nxt] = pltpu.make_async_copy(
                src(nxt), bufs[nxt % _NBUF], gsem.at[nxt % _NBUF]
            )
            ins[nxt].start()
    for c in range(n):
        if not out_waited[c]:
            outs[c].wait()


def kernel(new_k, new_v):
    shape = new_k.shape
    rows = new_k.size // 128
    k2 = new_k.reshape(rows, 128)
    v2 = new_v.reshape(rows, 128)
    out2 = pl.pallas_call(
        _copy_body,
        in_specs=[
            pl.BlockSpec(memory_space=pl.ANY),
            pl.BlockSpec(memory_space=pl.ANY),
        ],
        out_specs=[
            pl.BlockSpec(memory_space=pl.ANY),
            pl.BlockSpec(memory_space=pl.ANY),
        ],
        out_shape=(
            jax.ShapeDtypeStruct((rows, 128), jnp.float32),
            jax.ShapeDtypeStruct((rows, 128), jnp.float32),
        ),
        scratch_shapes=(
            [pltpu.VMEM((_CHUNK_ROWS, 128), jnp.float32)] * _NBUF
            + [pltpu.SemaphoreType.DMA((_NBUF,)), pltpu.SemaphoreType.DMA((_NBUF,))]
        ),
    )(k2, v2)
    return (out2[0].reshape(shape), out2[1].reshape(shape))


# speedup vs baseline: 1.1560x; 1.0013x over previous
import jax
import jax.numpy as jnp
from jax.experimental import pallas as pl
from jax.experimental.pallas import tpu as pltpu

_CHUNK_ROWS = 32768
_NBUF = 3
_LOOKAHEAD = 2


def _copy_body(k_ref, v_ref, ok_ref, ov_ref, *scratch):
    bufs = scratch[:_NBUF]
    gsem = scratch[_NBUF]
    ssem = scratch[_NBUF + 1]
    rows = k_ref.shape[0]
    per_tensor = rows // _CHUNK_ROWS
    n = 2 * per_tensor

    def src(c):
        ref = k_ref if c < per_tensor else v_ref
        return ref.at[pl.ds((c % per_tensor) * _CHUNK_ROWS, _CHUNK_ROWS)]

    def dst(c):
        ref = ok_ref if c < per_tensor else ov_ref
        return ref.at[pl.ds((c % per_tensor) * _CHUNK_ROWS, _CHUNK_ROWS)]

    ins = [None] * n
    outs = [None] * n
    out_waited = [False] * n

    for c in range(min(_LOOKAHEAD, n)):
        ins[c] = pltpu.make_async_copy(src(c), bufs[c % _NBUF], gsem.at[c % _NBUF])
        ins[c].start()
    for c in range(n):
        ins[c].wait()
        outs[c] = pltpu.make_async_copy(bufs[c % _NBUF], dst(c), ssem.at[c % _NBUF])
        outs[c].start()
        nxt = c + _LOOKAHEAD
        if nxt < n:
            if nxt >= _NBUF:
                outs[nxt - _NBUF].wait()
                out_waited[nxt - _NBUF] = True
            ins[nxt] = pltpu.make_async_copy(
                src(nxt), bufs[nxt % _NBUF], gsem.at[nxt % _NBUF]
            )
            ins[nxt].start()
    for c in range(n):
        if not out_waited[c]:
            outs[c].wait()


def kernel(new_k, new_v):
    shape = new_k.shape
    rows = new_k.size // 128
    k2 = new_k.reshape(rows, 128)
    v2 = new_v.reshape(rows, 128)
    out2 = pl.pallas_call(
        _copy_body,
        in_specs=[
            pl.BlockSpec(memory_space=pl.ANY),
            pl.BlockSpec(memory_space=pl.ANY),
        ],
        out_specs=[
            pl.BlockSpec(memory_space=pl.ANY),
            pl.BlockSpec(memory_space=pl.ANY),
        ],
        out_shape=(
            jax.ShapeDtypeStruct((rows, 128), jnp.float32),
            jax.ShapeDtypeStruct((rows, 128), jnp.float32),
        ),
        scratch_shapes=(
            [pltpu.VMEM((_CHUNK_ROWS, 128), jnp.float32)] * _NBUF
            + [pltpu.SemaphoreType.DMA((_NBUF,)), pltpu.SemaphoreType.DMA((_NBUF,))]
        ),
    )(k2, v2)
    return (out2[0].reshape(shape), out2[1].reshape(shape))


# speedup vs baseline: 1.1620x; 1.0052x over previous
import jax
import jax.numpy as jnp
from jax.experimental import pallas as pl
from jax.experimental.pallas import tpu as pltpu

_MAX_ROWS = 32768
_NBUF = 3
_LOOKAHEAD = 2

_SIZES = [8192] + [32768] * 7 + [16384] + [8192]


def _chunks(per_tensor_rows):
    assert sum(_SIZES) == per_tensor_rows
    seq = []
    for t in range(2):
        off = 0
        for s in _SIZES:
            seq.append((t, off, s))
            off += s
    return seq


def _copy_body(k_ref, v_ref, ok_ref, ov_ref, *scratch):
    bufs = scratch[:_NBUF]
    gsem = scratch[_NBUF]
    ssem = scratch[_NBUF + 1]
    seq = _chunks(k_ref.shape[0])
    n = len(seq)

    def src(c):
        t, off, s = seq[c]
        return (k_ref, v_ref)[t].at[pl.ds(off, s)]

    def dst(c):
        t, off, s = seq[c]
        return (ok_ref, ov_ref)[t].at[pl.ds(off, s)]

    def buf(c):
        s = seq[c][2]
        b = bufs[c % _NBUF]
        if s == _MAX_ROWS:
            return b
        return b.at[pl.ds(0, s)]

    ins = [None] * n
    outs = [None] * n
    out_waited = [False] * n

    for c in range(min(_LOOKAHEAD, n)):
        ins[c] = pltpu.make_async_copy(src(c), buf(c), gsem.at[c % _NBUF])
        ins[c].start()
    for c in range(n):
        ins[c].wait()
        outs[c] = pltpu.make_async_copy(buf(c), dst(c), ssem.at[c % _NBUF])
        outs[c].start()
        nxt = c + _LOOKAHEAD
        if nxt < n:
            if nxt >= _NBUF:
                outs[nxt - _NBUF].wait()
                out_waited[nxt - _NBUF] = True
            ins[nxt] = pltpu.make_async_copy(src(nxt), buf(nxt), gsem.at[nxt % _NBUF])
            ins[nxt].start()
    for c in range(n):
        if not out_waited[c]:
            outs[c].wait()


def kernel(new_k, new_v):
    shape = new_k.shape
    rows = new_k.size // 128
    k2 = new_k.reshape(rows, 128)
    v2 = new_v.reshape(rows, 128)
    out2 = pl.pallas_call(
        _copy_body,
        in_specs=[
            pl.BlockSpec(memory_space=pl.ANY),
            pl.BlockSpec(memory_space=pl.ANY),
        ],
        out_specs=[
            pl.BlockSpec(memory_space=pl.ANY),
            pl.BlockSpec(memory_space=pl.ANY),
        ],
        out_shape=(
            jax.ShapeDtypeStruct((rows, 128), jnp.float32),
            jax.ShapeDtypeStruct((rows, 128), jnp.float32),
        ),
        scratch_shapes=(
            [pltpu.VMEM((_MAX_ROWS, 128), jnp.float32)] * _NBUF
            + [pltpu.SemaphoreType.DMA((_NBUF,)), pltpu.SemaphoreType.DMA((_NBUF,))]
        ),
    )(k2, v2)
    return (out2[0].reshape(shape), out2[1].reshape(shape))
